# baseline (speedup 1.0000x reference)
import functools

import jax
import jax.numpy as jnp
from jax import lax
from jax.experimental import pallas as pl
from jax.experimental.pallas import tpu as pltpu
from jax.experimental.pallas import tpu_sc as plsc

N_ITEMS = 1000000
K = 32
BATCH = 16384
_RPL = 4
_NL = N_ITEMS // _RPL
_LW = _RPL * K

_info = plsc.get_sparse_core_info()
_NC = _info.num_cores
_NS = _info.num_subcores
_L = _info.num_lanes
_NW = _NC * _NS
_BPW = BATCH // _NW
_CH = 256

_mesh = plsc.VectorSubcoreMesh(core_axis_name="c", subcore_axis_name="s")


@functools.partial(
    pl.kernel,
    mesh=_mesh,
    out_type=jax.ShapeDtypeStruct((BATCH,), jnp.float32),
    scratch_types=[
        pltpu.VMEM((_BPW,), jnp.int32),
        pltpu.VMEM((_BPW,), jnp.int32),
        pltpu.VMEM((_BPW,), jnp.int32),
        pltpu.VMEM((_BPW,), jnp.int32),
        pltpu.VMEM((_CH, _LW), jnp.float32),
        pltpu.VMEM((_CH, _LW), jnp.float32),
        pltpu.VMEM((_BPW,), jnp.float32),
        pltpu.SemaphoreType.DMA,
        pltpu.SemaphoreType.DMA,
    ],
    compiler_params=pltpu.CompilerParams(needs_layout_passes=False),
)
def _mf_dot(u_idx_hbm, v_idx_hbm, u_hbm, v_hbm, out_hbm,
            uln_v, vln_v, urem_v, vrem_v, ulines_v, vlines_v, out_v,
            sem_u, sem_v):
    wid = lax.axis_index("s") * _NC + lax.axis_index("c")
    base = wid * _BPW

    pltpu.sync_copy(u_idx_hbm.at[pl.ds(base, _BPW)], uln_v)
    pltpu.sync_copy(v_idx_hbm.at[pl.ds(base, _BPW)], vln_v)

    def split(i, carry):
        ui = uln_v[pl.ds(i * _L, _L)]
        vi = vln_v[pl.ds(i * _L, _L)]
        urem_v[pl.ds(i * _L, _L)] = lax.rem(ui, _RPL) * K
        vrem_v[pl.ds(i * _L, _L)] = lax.rem(vi, _RPL) * K
        uln_v[pl.ds(i * _L, _L)] = lax.div(ui, _RPL)
        vln_v[pl.ds(i * _L, _L)] = lax.div(vi, _RPL)
        return carry

    lax.fori_loop(0, _BPW // _L, split, 0)

    def chunk(c, carry):
        off = c * _CH
        cp_u = pltpu.async_copy(
            u_hbm.at[uln_v.at[pl.ds(off, _CH)]], ulines_v, sem_u)
        cp_v = pltpu.async_copy(
            v_hbm.at[vln_v.at[pl.ds(off, _CH)]], vlines_v, sem_v)
        cp_u.wait()
        cp_v.wait()

        def group(g, carry2):
            gb = g * _L
            elems = lax.iota(jnp.int32, _L) + gb
            uc0 = urem_v[pl.ds(off + gb, _L)]
            vc0 = vrem_v[pl.ds(off + gb, _L)]
            acc = jnp.zeros((_L,), jnp.float32)
            for j in range(K):
                a = plsc.load_gather(ulines_v, [elems, uc0 + j])
                b = plsc.load_gather(vlines_v, [elems, vc0 + j])
                acc = acc + a * b
            out_v[pl.ds(off + gb, _L)] = acc
            return carry2

        lax.fori_loop(0, _CH // _L, group, 0)
        return carry

    lax.fori_loop(0, _BPW // _CH, chunk, 0)

    pltpu.sync_copy(out_v, out_hbm.at[pl.ds(base, _BPW)])


def kernel(u_idx, v_idx, U, V):
    u2 = U.reshape(_NL, _LW)
    v2 = V.reshape(_NL, _LW)
    return _mf_dot(u_idx.astype(jnp.int32), v_idx.astype(jnp.int32), u2, v2)
